# Initial kernel scaffold; baseline (speedup 1.0000x reference)
import jax
import jax.numpy as jnp
from jax.experimental import pallas as pl


def kernel(prior, rates):
    raise NotImplementedError("write your pallas kernel here")


# speedup vs baseline: 48.5063x; 48.5063x over previous
import functools

import numpy as np
import jax
import jax.numpy as jnp
from jax import lax
from jax.experimental import pallas as pl
from jax.experimental.pallas import tpu as pltpu
from jax.experimental.pallas import tpu_sc as plsc

L = 16
IMIN = np.int32(-2147483648)

H1 = 2048
H2 = 2048
H3 = 1024


def _make_kernel(B, N, n_workers):
    rows_per_w = B // n_workers
    n_vec = N // L
    mesh = plsc.VectorSubcoreMesh(core_axis_name="c", subcore_axis_name="s")

    @functools.partial(
        pl.kernel,
        mesh=mesh,
        out_type=jax.ShapeDtypeStruct((B, N), jnp.int32),
        scratch_types=[
            pltpu.VMEM((N,), jnp.float32),
            pltpu.VMEM((N,), jnp.int32),
            pltpu.VMEM((H1 * L,), jnp.int32),
            pltpu.VMEM((B,), jnp.float32),
        ],
        compiler_params=pltpu.CompilerParams(needs_layout_passes=False),
    )
    def masksel(prior_hbm, rates_hbm, out_hbm, rowbuf, maskbuf, hist, ratebuf):
        wid = lax.axis_index("c") * 16 + lax.axis_index("s")
        iota = lax.iota(jnp.int32, L)
        zeros = iota & 0
        ones = zeros + 1

        pltpu.sync_copy(rates_hbm, ratebuf)

        def clear_hist(n_rows):
            def body(i, c):
                hist[pl.ds(i * L, L)] = zeros
                return c
            lax.fori_loop(0, n_rows, body, 0)

        def cumsum_hist(n_rows):
            def body(i, acc):
                v = hist[pl.ds(i * L, L)]
                acc = acc + v
                hist[pl.ds(i * L, L)] = acc
                return acc
            lax.fori_loop(0, n_rows, body, zeros)

        def csum_at(b):
            return jnp.sum(hist[pl.ds(b * L, L)])

        def search(base, k, n_rows):
            pos = k * 0
            step = n_rows // 2
            while step >= 1:
                c = csum_at(pos + (step - 1))
                pos = pos + jnp.where(base + c < k, np.int32(step), np.int32(0))
                step //= 2
            below = jnp.where(pos > 0,
                              jnp.sum(hist[pl.ds((jnp.maximum(pos, 1) - 1) * L, L)]),
                              np.int32(0))
            return pos, base + below

        def do_row(rr, _):
            pltpu.sync_copy(prior_hbm.at[rr], rowbuf)

            rv = ratebuf[pl.ds((rr >> 4) * L, L)]
            kv = (rv * np.float32(N)).astype(jnp.int32)
            k = jnp.sum(jnp.where(iota == (rr & 15), kv, 0))

            clear_hist(H1)

            def pass1(i, c):
                v = rowbuf[pl.ds(i * L, L)]
                b = plsc.bitcast(v, jnp.int32)
                key = jnp.where(b >= 0, b, IMIN - b)
                rowbuf[pl.ds(i * L, L)] = plsc.bitcast(key, jnp.float32)
                bb = (key >> 21) + 1024
                plsc.addupdate_scatter(hist, [bb * L + iota], ones)
                return c
            lax.fori_loop(0, n_vec, pass1, 0)
            cumsum_hist(H1)
            b1, base1 = search(np.int32(0), k, H1)

            clear_hist(H2)

            def pass2(i, c):
                key = plsc.bitcast(rowbuf[pl.ds(i * L, L)], jnp.int32)
                m = ((key >> 21) + 1024) == b1
                bb = (key >> 10) & 0x7FF
                plsc.addupdate_scatter(hist, [bb * L + iota], m.astype(jnp.int32))
                return c
            lax.fori_loop(0, n_vec, pass2, 0)
            cumsum_hist(H2)
            b2, base2 = search(base1, k, H2)

            p2 = ((b1 - 1024) << 11) | b2
            clear_hist(H3)

            def pass3(i, c):
                key = plsc.bitcast(rowbuf[pl.ds(i * L, L)], jnp.int32)
                m = (key >> 10) == p2
                bb = key & 0x3FF
                plsc.addupdate_scatter(hist, [bb * L + iota], m.astype(jnp.int32))
                return c
            lax.fori_loop(0, n_vec, pass3, 0)
            cumsum_hist(H3)
            b3, c_less = search(base2, k, H3)

            T = (p2 << 10) | b3
            r = k - c_less

            def pass4(i, run):
                key = plsc.bitcast(rowbuf[pl.ds(i * L, L)], jnp.int32)
                m_lt = key < T
                e = (key == T).astype(jnp.int32)
                pfx = jnp.cumsum(e)
                sel = m_lt | ((e > 0) & ((run + pfx) <= r))
                maskbuf[pl.ds(i * L, L)] = sel.astype(jnp.int32)
                return run + jnp.sum(e)
            lax.fori_loop(0, n_vec, pass4, np.int32(0))

            pltpu.sync_copy(maskbuf, out_hbm.at[rr])
            return _

        lax.fori_loop(wid * rows_per_w, (wid + 1) * rows_per_w, do_row, 0)

    return masksel


def kernel(prior, rates):
    B, N = prior.shape
    out = _make_kernel(B, N, 32)(prior, rates.reshape(B))
    return out.astype(bool)


# speedup vs baseline: 64.9901x; 1.3398x over previous
import functools

import numpy as np
import jax
import jax.numpy as jnp
from jax import lax
from jax.experimental import pallas as pl
from jax.experimental.pallas import tpu as pltpu
from jax.experimental.pallas import tpu_sc as plsc

L = 16
IMIN = np.int32(-2147483648)

HB = 256
REP = 4
UNROLL = 4


def _make_kernel(B, N, n_workers):
    rows_per_w = B // n_workers
    n_vec = N // L
    n_it = n_vec // UNROLL
    mesh = plsc.VectorSubcoreMesh(core_axis_name="c", subcore_axis_name="s")

    @functools.partial(
        pl.kernel,
        mesh=mesh,
        out_type=jax.ShapeDtypeStruct((B, N), jnp.int32),
        scratch_types=[
            pltpu.VMEM((N,), jnp.float32),
            pltpu.VMEM((N,), jnp.int32),
            pltpu.VMEM((REP * HB * L,), jnp.int32),
            pltpu.VMEM((B,), jnp.float32),
        ],
        compiler_params=pltpu.CompilerParams(needs_layout_passes=False),
    )
    def masksel(prior_hbm, rates_hbm, out_hbm, rowbuf, maskbuf, hist, ratebuf):
        wid = lax.axis_index("c") * 16 + lax.axis_index("s")
        iota = lax.iota(jnp.int32, L)
        zeros = iota & 0
        ones = zeros + 1
        iota_rep = [iota + u * (HB * L) for u in range(REP)]

        pltpu.sync_copy(rates_hbm, ratebuf)

        def clear_hist():
            def body(i, c):
                for u in range(UNROLL):
                    hist[pl.ds((i * UNROLL + u) * L, L)] = zeros
                return c
            lax.fori_loop(0, REP * HB // UNROLL, body, 0)

        def cumsum_hist():
            def body(i, acc):
                v = hist[pl.ds(i * L, L)]
                for u in range(1, REP):
                    v = v + hist[pl.ds((u * HB + i) * L, L)]
                acc = acc + v
                hist[pl.ds(i * L, L)] = acc
                return acc
            lax.fori_loop(0, HB, body, zeros)

        def csum_at(b):
            return jnp.sum(hist[pl.ds(b * L, L)])

        def search(base, k):
            pos = k * 0
            step = HB // 2
            while step >= 1:
                c = csum_at(pos + (step - 1))
                pos = pos + jnp.where(base + c < k, np.int32(step), np.int32(0))
                step //= 2
            below = jnp.where(pos > 0,
                              jnp.sum(hist[pl.ds((jnp.maximum(pos, 1) - 1) * L, L)]),
                              np.int32(0))
            return pos, base + below

        def do_row(rr, _):
            pltpu.sync_copy(prior_hbm.at[rr], rowbuf)

            rv = ratebuf[pl.ds((rr >> 4) * L, L)]
            kv = (rv * np.float32(N)).astype(jnp.int32)
            k = jnp.sum(jnp.where(iota == (rr & 15), kv, 0))

            clear_hist()

            def pass1(i, c):
                for u in range(UNROLL):
                    j = i * UNROLL + u
                    v = rowbuf[pl.ds(j * L, L)]
                    b = plsc.bitcast(v, jnp.int32)
                    key = jnp.where(b >= 0, b, IMIN - b)
                    rowbuf[pl.ds(j * L, L)] = plsc.bitcast(key, jnp.float32)
                    bb = (key >> 24) + 128
                    plsc.addupdate_scatter(hist, [(bb << 4) + iota_rep[u]], ones)
                return c
            lax.fori_loop(0, n_it, pass1, 0)
            cumsum_hist()
            b1, base1 = search(k * 0, k)
            p1 = b1 - 128

            clear_hist()

            def pass2(i, c):
                for u in range(UNROLL):
                    j = i * UNROLL + u
                    key = plsc.bitcast(rowbuf[pl.ds(j * L, L)], jnp.int32)
                    m = (key >> 24) == p1
                    bb = (key >> 16) & 0xFF
                    plsc.addupdate_scatter(hist, [(bb << 4) + iota_rep[u]],
                                           m.astype(jnp.int32))
                return c
            lax.fori_loop(0, n_it, pass2, 0)
            cumsum_hist()
            b2, base2 = search(base1, k)
            p2 = (p1 << 8) | b2

            clear_hist()

            def pass3(i, c):
                for u in range(UNROLL):
                    j = i * UNROLL + u
                    key = plsc.bitcast(rowbuf[pl.ds(j * L, L)], jnp.int32)
                    m = (key >> 16) == p2
                    bb = (key >> 8) & 0xFF
                    plsc.addupdate_scatter(hist, [(bb << 4) + iota_rep[u]],
                                           m.astype(jnp.int32))
                return c
            lax.fori_loop(0, n_it, pass3, 0)
            cumsum_hist()
            b3, base3 = search(base2, k)
            p3 = (p2 << 8) | b3

            clear_hist()

            def pass4(i, c):
                for u in range(UNROLL):
                    j = i * UNROLL + u
                    key = plsc.bitcast(rowbuf[pl.ds(j * L, L)], jnp.int32)
                    m = (key >> 8) == p3
                    bb = key & 0xFF
                    plsc.addupdate_scatter(hist, [(bb << 4) + iota_rep[u]],
                                           m.astype(jnp.int32))
                return c
            lax.fori_loop(0, n_it, pass4, 0)
            cumsum_hist()
            b4, c_less = search(base3, k)

            T = (p3 << 8) | b4
            c_leq = base3 + csum_at(b4)
            cnt_eq = c_leq - c_less
            r = k - c_less

            def mask_fast(i, c):
                for u in range(UNROLL):
                    j = i * UNROLL + u
                    key = plsc.bitcast(rowbuf[pl.ds(j * L, L)], jnp.int32)
                    maskbuf[pl.ds(j * L, L)] = (key <= T).astype(jnp.int32)
                return c
            lax.fori_loop(0, n_it, mask_fast, 0)

            @pl.when(r < cnt_eq)
            def _tie_fixup():
                def mask_exact(i, run):
                    key = plsc.bitcast(rowbuf[pl.ds(i * L, L)], jnp.int32)
                    m_lt = key < T
                    e = (key == T).astype(jnp.int32)
                    pfx = jnp.cumsum(e)
                    sel = m_lt | ((e > 0) & ((run + pfx) <= r))
                    maskbuf[pl.ds(i * L, L)] = sel.astype(jnp.int32)
                    return run + jnp.sum(e)
                lax.fori_loop(0, n_vec, mask_exact, k * 0)

            pltpu.sync_copy(maskbuf, out_hbm.at[rr])
            return _

        lax.fori_loop(wid * rows_per_w, (wid + 1) * rows_per_w, do_row, 0)

    return masksel


def kernel(prior, rates):
    B, N = prior.shape
    out = _make_kernel(B, N, 32)(prior, rates.reshape(B))
    return out.astype(bool)
